# baseline (speedup 1.0000x reference)
import functools

import jax
import jax.numpy as jnp
from jax import lax
from jax.experimental import pallas as pl
from jax.experimental.pallas import tpu as pltpu
from jax.experimental.pallas import tpu_sc as plsc

F32 = jnp.float32
N_NODE = 10000
HID = 128
CHUNK = 128
NWORK = 32
NODE_BLK = 2000
EDGE_BLK = 2048


def _tables_body(pos_s_ref, h_ref, x_ref, pos_a_ref, u_ref,
                 wp_ref, wh_ref, wx_ref, bs_ref, wpa_ref, wua_ref,
                 tadst_ref, tssrc_ref, tsdst_ref, tasrc_ref):
    pos_s = pos_s_ref[...]
    wp = wp_ref[...]
    r = (pos_s[:, 0:1] * wp[0:1, :] + pos_s[:, 1:2] * wp[1:2, :]
         + jnp.dot(h_ref[...], wh_ref[...], preferred_element_type=F32)
         + jnp.dot(x_ref[...], wx_ref[...], preferred_element_type=F32)
         + bs_ref[...])
    tadst_ref[...] = r[:, 0:HID]
    tssrc_ref[...] = r[:, HID:2 * HID]
    tsdst_ref[...] = r[:, 2 * HID:3 * HID]
    pos_a = pos_a_ref[...]
    wpa = wpa_ref[...]
    tasrc_ref[...] = (pos_a[:, 0:1] * wpa[0:1, :] + pos_a[:, 1:2] * wpa[1:2, :]
                      + jnp.dot(u_ref[...], wua_ref[...], preferred_element_type=F32))


def _node_tables(pos_s, h, x, pos_a, u, wp, wh, wx, bs, wpa, wua):
    n = pos_s.shape[0]
    grid = (n // NODE_BLK,)
    row = lambda w: pl.BlockSpec((NODE_BLK, w), lambda i: (i, 0))
    full = lambda a, b: pl.BlockSpec((a, b), lambda i: (0, 0))
    return pl.pallas_call(
        _tables_body,
        grid=grid,
        in_specs=[row(2), row(HID), row(HID), row(2), row(HID),
                  full(2, 3 * HID), full(HID, 3 * HID), full(HID, 3 * HID),
                  full(1, 3 * HID), full(2, HID), full(HID, HID)],
        out_specs=[row(HID), row(HID), row(HID), row(HID)],
        out_shape=[jax.ShapeDtypeStruct((n, HID), F32)] * 4,
    )(pos_s, h, x, pos_a, u, wp, wh, wx, bs, wpa, wua)


def _add_chunk(bs, bd, bz):
    def inner(r, carry):
        for rr in range(2):
            for j in range(HID // 16):
                sl = pl.ds(j * 16, 16)
                bz[r * 2 + rr, sl] = bs[r * 2 + rr, sl] + bd[r * 2 + rr, sl]
        return carry

    lax.fori_loop(0, CHUNK // 2, inner, 0)


def _gather_add_body(sidx, didx, tsrc, tdst, z_hbm,
                     is0, is1, id0, id1, bs0, bs1, bd0, bd1, bz0, bz1,
                     si0, si1, sg0, sg1, st0, st1):
    c = lax.axis_index("c")
    s = lax.axis_index("s")
    wid = s * 2 + c
    per = z_hbm.shape[0] // CHUNK // NWORK
    base_ch = wid * per
    isv = (is0, is1)
    idv = (id0, id1)
    bs = (bs0, bs1)
    bd = (bd0, bd1)
    bz = (bz0, bz1)
    si = (si0, si1)
    sg = (sg0, sg1)
    st = (st0, st1)

    def pair(jp, carry):
        ix = []
        for b in range(2):
            base_e = (base_ch + jp * 2 + b) * CHUNK
            ix.append(pltpu.async_copy(sidx.at[pl.ds(base_e, CHUNK)],
                                       isv[b], si[b]))
            ix.append(pltpu.async_copy(didx.at[pl.ds(base_e, CHUNK)],
                                       idv[b], si[b]))
        gd = []
        for b in range(2):
            ix[2 * b].wait()
            ix[2 * b + 1].wait()
            gd.append(pltpu.async_copy(tsrc.at[isv[b]], bs[b], sg[b]))
            gd.append(pltpu.async_copy(tdst.at[idv[b]], bd[b], sg[b]))
        sd = []
        for b in range(2):
            gd[2 * b].wait()
            gd[2 * b + 1].wait()
            _add_chunk(bs[b], bd[b], bz[b])
            base_e = (base_ch + jp * 2 + b) * CHUNK
            sd.append(pltpu.async_copy(bz[b], z_hbm.at[pl.ds(base_e, CHUNK)],
                                       st[b]))
        for d in sd:
            d.wait()
        return carry

    lax.fori_loop(0, per // 2, pair, 0)


def _gather_add(sidx, didx, tsrc, tdst):
    ea = sidx.shape[0]
    mesh = plsc.VectorSubcoreMesh(core_axis_name="c", subcore_axis_name="s")
    scratch = ([pltpu.VMEM((CHUNK,), jnp.int32)] * 4
               + [pltpu.VMEM((CHUNK, HID), F32)] * 6
               + [pltpu.SemaphoreType.DMA] * 6)
    fn = pl.kernel(
        _gather_add_body,
        out_type=jax.ShapeDtypeStruct((ea, HID), F32),
        mesh=mesh,
        scratch_types=scratch,
    )
    return fn(sidx, didx, tsrc, tdst)


def _edge_body(z_ref, dis_ref, wd_ref, w2_ref, b2_ref, w3_ref, b3_ref,
               num_ref, den_ref):
    z1 = jnp.tanh(z_ref[...] + dis_ref[...] * wd_ref[...])
    h2 = jnp.tanh(jnp.dot(z1, w2_ref[...], preferred_element_type=F32)
                  + b2_ref[...])
    o = jnp.dot(h2, w3_ref[...], preferred_element_type=F32) + b3_ref[...]
    el = jnp.exp(o[:, 0:HID])
    num_ref[...] = el * o[:, HID:2 * HID]
    den_ref[...] = el


def _edge_mlp(z, dis, wd, w2, b2, w3, b3):
    ea = z.shape[0]
    grid = (ea // EDGE_BLK,)
    row = lambda w: pl.BlockSpec((EDGE_BLK, w), lambda i: (i, 0))
    full = lambda a, b: pl.BlockSpec((a, b), lambda i: (0, 0))
    return pl.pallas_call(
        _edge_body,
        grid=grid,
        in_specs=[row(HID), row(1),
                  full(1, HID), full(HID, HID), full(1, HID),
                  full(HID, 2 * HID), full(1, 2 * HID)],
        out_specs=[row(HID), row(HID)],
        out_shape=[jax.ShapeDtypeStruct((ea, HID), F32)] * 2,
    )(z, dis, wd, w2, b2, w3, b3)


def _scatter_body(didx, num, den, zeros, out_n, out_d,
                  i0, i1, b0, b1, acc_sh, si0, si1, sl0, sl1, sa0, sa1):
    c = lax.axis_index("c")
    s = lax.axis_index("s")
    n_sub = 16
    per = didx.shape[0] // CHUNK // n_sub
    base_ch = s * per

    @pl.when(s == 0)
    def _():
        pltpu.sync_copy(zeros, acc_sh)

    plsc.subcore_barrier()
    idxs = (i0, i1)
    bufs = (b0, b1)
    sis = (si0, si1)
    sls = (sl0, sl1)
    sas = (sa0, sa1)

    def run(src_hbm):
        def pair(jp, carry):
            ld = []
            for b in range(2):
                base_e = (base_ch + jp * 2 + b) * CHUNK
                ld.append(pltpu.async_copy(didx.at[pl.ds(base_e, CHUNK)],
                                           idxs[b], sis[b]))
                ld.append(pltpu.async_copy(src_hbm.at[pl.ds(base_e, CHUNK)],
                                           bufs[b], sls[b]))
            ad = []
            for b in range(2):
                ld[2 * b].wait()
                ld[2 * b + 1].wait()
                ad.append(pltpu.async_copy(bufs[b], acc_sh.at[idxs[b]],
                                           sas[b], add=True))
            for d in ad:
                d.wait()
            return carry

        lax.fori_loop(0, per // 2, pair, 0)

    pl.when(c == 0)(lambda: run(num))
    pl.when(c == 1)(lambda: run(den))
    plsc.subcore_barrier()

    @pl.when(s < 10)
    def _():
        rows = pl.ds(s * 1000, 1000)
        pl.when(c == 0)(lambda: pltpu.sync_copy(acc_sh.at[rows],
                                                out_n.at[rows]))
        pl.when(c == 1)(lambda: pltpu.sync_copy(acc_sh.at[rows],
                                                out_d.at[rows]))


def _scatter_add(didx, num, den, zeros):
    mesh = plsc.VectorSubcoreMesh(core_axis_name="c", subcore_axis_name="s")
    scratch = [pltpu.VMEM((CHUNK,), jnp.int32),
               pltpu.VMEM((CHUNK,), jnp.int32),
               pltpu.VMEM((CHUNK, HID), F32),
               pltpu.VMEM((CHUNK, HID), F32),
               pltpu.VMEM_SHARED((N_NODE + 8, HID), F32),
               pltpu.SemaphoreType.DMA,
               pltpu.SemaphoreType.DMA,
               pltpu.SemaphoreType.DMA,
               pltpu.SemaphoreType.DMA,
               pltpu.SemaphoreType.DMA,
               pltpu.SemaphoreType.DMA]
    fn = pl.kernel(
        _scatter_body,
        out_type=[jax.ShapeDtypeStruct((N_NODE, HID), F32)] * 2,
        mesh=mesh,
        scratch_types=scratch,
    )
    return fn(didx, num, den, zeros)


def _final_body(pos_ref, h_ref, x_ref, numa_ref, dena_ref, nums_ref, dens_ref,
                wp_ref, wh_ref, wsu_ref, wsx_ref, wx2_ref, b1_ref,
                w2_ref, b2_ref, w3_ref, b3_ref, out_ref):
    dena = dena_ref[...]
    dens = dens_ref[...]
    sum_u = jnp.where(dena != 0, numa_ref[...] / dena, 0.0)
    sum_x = jnp.where(dens != 0, nums_ref[...] / dens, 0.0)
    pos = pos_ref[...]
    wp = wp_ref[...]
    t1 = jnp.tanh(
        pos[:, 0:1] * wp[0:1, :] + pos[:, 1:2] * wp[1:2, :]
        + jnp.dot(h_ref[...], wh_ref[...], preferred_element_type=F32)
        + jnp.dot(sum_u, wsu_ref[...], preferred_element_type=F32)
        + jnp.dot(sum_x, wsx_ref[...], preferred_element_type=F32)
        + jnp.dot(x_ref[...], wx2_ref[...], preferred_element_type=F32)
        + b1_ref[...])
    t2 = jnp.tanh(jnp.dot(t1, w2_ref[...], preferred_element_type=F32)
                  + b2_ref[...])
    out_ref[...] = (jnp.dot(t2, w3_ref[...], preferred_element_type=F32)
                    + b3_ref[...])


def _final_mlp(pos_s, h, x, numa, dena, nums, dens, wp, wh, wsu, wsx, wx2, b1,
               w2, b2, w3, b3):
    n = pos_s.shape[0]
    grid = (n // NODE_BLK,)
    row = lambda w: pl.BlockSpec((NODE_BLK, w), lambda i: (i, 0))
    full = lambda a, b: pl.BlockSpec((a, b), lambda i: (0, 0))
    mlp = 64
    return pl.pallas_call(
        _final_body,
        grid=grid,
        in_specs=[row(2), row(HID), row(HID), row(HID), row(HID), row(HID),
                  row(HID),
                  full(2, mlp), full(HID, mlp), full(HID, mlp),
                  full(HID, mlp), full(HID, mlp), full(1, mlp),
                  full(mlp, mlp), full(1, mlp), full(mlp, HID),
                  full(1, HID)],
        out_specs=row(HID),
        out_shape=jax.ShapeDtypeStruct((n, HID), F32),
    )(pos_s, h, x, numa, dena, nums, dens, wp, wh, wsu, wsx, wx2, b1, w2, b2,
      w3, b3)


def _fuse_heads(pa, pb):
    w1 = jnp.concatenate([pa["W1"], pb["W1"]], axis=1)
    b1 = jnp.concatenate([pa["b1"], pb["b1"]])
    z = jnp.zeros_like(pa["W2"])
    w2 = jnp.concatenate([jnp.concatenate([pa["W2"], z], 1),
                          jnp.concatenate([z, pb["W2"]], 1)], 0)
    b2 = jnp.concatenate([pa["b2"], pb["b2"]])
    z3 = jnp.zeros_like(pa["W3"])
    w3 = jnp.concatenate([jnp.concatenate([pa["W3"], z3], 1),
                          jnp.concatenate([z3, pb["W3"]], 1)], 0)
    b3 = jnp.concatenate([pa["b3"], pb["b3"]])
    return w1, b1, w2, b2, w3, b3


def kernel(h, x, u, pos_state, pos_action, dis_a2s, dis_s2s, edge_a2s,
           edge_s2s, params):
    f = HID
    w1u, b1u, w2u, b2u, w3u, b3u = _fuse_heads(params["u2h_logit"],
                                               params["u2h_u"])
    w1x, b1x, w2x, b2x, w3x, b3x = _fuse_heads(params["x2h_logit"],
                                               params["x2h_x"])
    wp = jnp.concatenate([w1u[2:4], w1x[0:2], w1x[2:4]], axis=1)
    wh = jnp.concatenate([w1u[133:261], w1x[5:133], w1x[261:389]], 1)
    wx = jnp.concatenate([w1u[261:389], w1x[133:261], w1x[389:517]], 1)
    bs = jnp.concatenate([b1u, jnp.zeros_like(b1x), b1x]).reshape(1, 3 * f)
    wpa = w1u[0:2]
    wua = w1u[5:133]
    ta_dst, ts_src, ts_dst, ta_src = _node_tables(
        pos_state, h, x, pos_action, u, wp, wh, wx, bs, wpa, wua)

    ea_raw = edge_a2s.shape[1]
    ea_pad = -(-ea_raw // (NWORK * CHUNK)) * (NWORK * CHUNK)
    pad = ea_pad - ea_raw

    def pad_idx(v, fill):
        return jnp.concatenate(
            [v.astype(jnp.int32), jnp.full((pad,), fill, jnp.int32)])

    src_a2d = pad_idx(edge_a2s[0], 0)
    dst_a2d = pad_idx(edge_a2s[1], N_NODE)
    src_s2d = pad_idx(edge_s2s[0], 0)
    dst_s2d = pad_idx(edge_s2s[1], N_NODE)
    dis_a = jnp.concatenate([dis_a2s, jnp.zeros((pad, 1), F32)])
    dis_s = jnp.concatenate([dis_s2s, jnp.zeros((pad, 1), F32)])

    z_a = _gather_add(src_a2d, dst_a2d, ta_src, ta_dst)
    z_s = _gather_add(src_s2d, dst_s2d, ts_src, ts_dst)

    num_a, den_a = _edge_mlp(z_a, dis_a, w1u[4:5], w2u,
                             b2u.reshape(1, 2 * 64), w3u,
                             b3u.reshape(1, 2 * f))
    num_s, den_s = _edge_mlp(z_s, dis_s, w1x[4:5], w2x,
                             b2x.reshape(1, 2 * 64), w3x,
                             b3x.reshape(1, 2 * f))

    zeros = jnp.zeros((N_NODE + 8, f), F32)
    numa, dena = _scatter_add(dst_a2d, num_a, den_a, zeros)
    nums, dens = _scatter_add(dst_s2d, num_s, den_s, zeros)

    pu = params["h_updater"]
    w1f = pu["W1"]
    return _final_mlp(
        pos_state, h, x, numa, dena, nums, dens,
        w1f[0:2], w1f[2:130], w1f[130:258], w1f[258:386], w1f[386:514],
        pu["b1"].reshape(1, -1), pu["W2"], pu["b2"].reshape(1, -1),
        pu["W3"], pu["b3"].reshape(1, -1))
